# baseline (speedup 1.0000x reference)
import functools

import jax
import jax.numpy as jnp
from jax import lax
from jax.experimental import pallas as pl
from jax.experimental.pallas import tpu as pltpu
from jax.experimental.pallas import tpu_sc as plsc

S, D, FF, E = 2048, 1024, 2048, 8
TILE = 128
NT = 24
NPAD = NT * TILE
NW = 32
CHUNK = S // NW
FFC = 1024
NF = FF // FFC
NBUF = 3


def _router_body(x_ref, rw_ref, logits_ref, ei_ref, xs_ref, pos_ref, te_ref):
    x = x_ref[...]
    logits = lax.dot_general(
        x, rw_ref[...], (((1,), (0,)), ((), ())),
        precision=lax.Precision.DEFAULT, preferred_element_type=jnp.float32)
    logits_ref[...] = logits
    m = jnp.max(logits, axis=1, keepdims=True)
    ex = jnp.exp(logits - m)
    probs = ex / jnp.sum(ex, axis=1, keepdims=True)
    pmax = jnp.max(probs, axis=1, keepdims=True)
    col = lax.broadcasted_iota(jnp.int32, (S, E), 1)
    ei = jnp.min(jnp.where(probs == pmax, col, E), axis=1, keepdims=True)
    ei_ref[...] = ei
    xs_ref[...] = x * pmax
    onehot = (col == ei).astype(jnp.bfloat16)
    tri = (lax.broadcasted_iota(jnp.int32, (S, S), 1)
           <= lax.broadcasted_iota(jnp.int32, (S, S), 0)).astype(jnp.bfloat16)
    ranks = lax.dot_general(tri, onehot, (((1,), (0,)), ((), ())),
                            preferred_element_type=jnp.float32)
    counts = ranks[S - 1:S, :].astype(jnp.int32)
    ntiles = (counts + (TILE - 1)) // TILE
    etri = (lax.broadcasted_iota(jnp.int32, (E, E), 0)
            < lax.broadcasted_iota(jnp.int32, (E, E), 1)).astype(jnp.bfloat16)
    cum_excl = lax.dot_general(ntiles.astype(jnp.bfloat16), etri,
                               (((1,), (0,)), ((), ())),
                               preferred_element_type=jnp.float32)
    row_off = cum_excl * float(TILE)
    onehot_f = onehot.astype(jnp.float32)
    pos = jnp.sum(onehot_f * (row_off + ranks - 1.0), axis=1, keepdims=True)
    pos_ref[...] = pos.astype(jnp.int32)
    cum_incl = cum_excl + ntiles.astype(jnp.float32)
    te_ref[...] = jnp.concatenate(
        [cum_excl.astype(jnp.int32), cum_incl.astype(jnp.int32)], axis=0)


def _ffn_body(meta_ref, x_ref, pos_ref, wi_hbm, wo_hbm, y_ref, wi_b, wo_b,
              xt_b, sems):
    copies = {}
    posc = pos_ref[...]

    def start(j):
        e, f = divmod(j, NF)
        slot = j % NBUF
        c1 = pltpu.make_async_copy(
            wi_hbm.at[e, :, pl.ds(f * FFC, FFC)], wi_b.at[slot],
            sems.at[2 * slot])
        c2 = pltpu.make_async_copy(
            wo_hbm.at[e, pl.ds(f * FFC, FFC), :], wo_b.at[slot],
            sems.at[2 * slot + 1])
        c1.start()
        c2.start()
        copies[j] = (c1, c2)

    for j in range(NBUF - 1):
        start(j)
    for j in range(E * NF):
        e, f = divmod(j, NF)
        slot = j % NBUF
        if j + NBUF - 1 < E * NF:
            start(j + NBUF - 1)
        c1, c2 = copies.pop(j)
        c1.wait()
        c2.wait()
        t0 = meta_ref[0, e]
        t1 = meta_ref[1, e]

        def tile_body(t, carry, slot=slot, f=f):
            row = pl.multiple_of(t * TILE, TILE)
            lrow = pl.multiple_of((t - t0) * TILE, TILE)
            if f == 0:
                slot_ids = lax.broadcasted_iota(jnp.int32, (S, TILE), 1) + row
                pt = (posc == slot_ids).astype(jnp.float32)
                xt = lax.dot_general(pt, x_ref[...], (((0,), (0,)), ((), ())),
                                     preferred_element_type=jnp.float32)
                xt_b[pl.ds(lrow, TILE), :] = xt.astype(jnp.bfloat16)
            else:
                xt = xt_b[pl.ds(lrow, TILE), :].astype(jnp.float32)
            h = jnp.maximum(
                jnp.dot(xt, wi_b[slot], preferred_element_type=jnp.float32),
                0.0)
            yv = jnp.dot(h, wo_b[slot], preferred_element_type=jnp.float32)
            if f == 0:
                y_ref[pl.ds(row, TILE), :] = yv
            else:
                y_ref[pl.ds(row, TILE), :] += yv
            return carry

        lax.fori_loop(t0, t1, tile_body, 0)


def _sc_gather_body(y_hbm, pos_hbm, out_hbm, idx_v, rows_v, sem):
    wid = lax.axis_index("s") * 2 + lax.axis_index("c")
    base = wid * CHUNK
    pltpu.sync_copy(pos_hbm.at[pl.ds(base, CHUNK)], idx_v)
    pltpu.async_copy(y_hbm.at[idx_v], rows_v, sem).wait()
    pltpu.sync_copy(rows_v, out_hbm.at[pl.ds(base, CHUNK)])


def kernel(hidden_states, router_weight, wi, wo):
    x2d = hidden_states.reshape(S, D)

    logits, ei, xs, pos, te = pl.pallas_call(
        _router_body,
        out_shape=(
            jax.ShapeDtypeStruct((S, E), jnp.float32),
            jax.ShapeDtypeStruct((S, 1), jnp.int32),
            jax.ShapeDtypeStruct((S, D), jnp.float32),
            jax.ShapeDtypeStruct((S, 1), jnp.int32),
            jax.ShapeDtypeStruct((2, E), jnp.int32),
        ),
    )(x2d, router_weight)

    pos1d = pos.reshape(S)

    mesh = plsc.VectorSubcoreMesh(core_axis_name="c", subcore_axis_name="s")
    y_sorted = pl.pallas_call(
        _ffn_body,
        grid_spec=pltpu.PrefetchScalarGridSpec(
            num_scalar_prefetch=1,
            grid=(1,),
            in_specs=[
                pl.BlockSpec((S, D), lambda t, m_s: (0, 0)),
                pl.BlockSpec((S, 1), lambda t, m_s: (0, 0)),
                pl.BlockSpec(memory_space=pltpu.MemorySpace.HBM),
                pl.BlockSpec(memory_space=pltpu.MemorySpace.HBM),
            ],
            out_specs=pl.BlockSpec((NPAD, D), lambda t, m_s: (0, 0)),
            scratch_shapes=[
                pltpu.VMEM((NBUF, D, FFC), jnp.float32),
                pltpu.VMEM((NBUF, FFC, D), jnp.float32),
                pltpu.VMEM((S, D), jnp.bfloat16),
                pltpu.SemaphoreType.DMA((2 * NBUF,)),
            ],
        ),
        out_shape=jax.ShapeDtypeStruct((NPAD, D), jnp.float32),
    )(te, xs, pos, wi, wo)

    next2d = pl.kernel(
        _sc_gather_body,
        out_type=jax.ShapeDtypeStruct((S, D), jnp.float32),
        mesh=mesh,
        scratch_types=[
            pltpu.VMEM((CHUNK,), jnp.int32),
            pltpu.VMEM((CHUNK, D), jnp.float32),
            pltpu.SemaphoreType.DMA,
        ],
    )(y_sorted, pos1d)

    return (next2d.reshape(1, S, D), logits.reshape(1, S, E), ei.reshape(1, S))


# speedup vs baseline: 1.1054x; 1.1054x over previous
import functools

import jax
import jax.numpy as jnp
from jax import lax
from jax.experimental import pallas as pl
from jax.experimental.pallas import tpu as pltpu
from jax.experimental.pallas import tpu_sc as plsc

S, D, FF, E = 2048, 1024, 2048, 8
TILE = 128
NT = 24
NPAD = NT * TILE
NW = 32
CHUNK = S // NW
FFC = 1024
NF = FF // FFC
NBUF = 3


def _router_body(x_ref, rw_ref, logits_ref, ei_ref, xs_ref, pos_ref, te_ref):
    x = x_ref[...]
    logits = lax.dot_general(
        x, rw_ref[...], (((1,), (0,)), ((), ())),
        precision=lax.Precision.DEFAULT, preferred_element_type=jnp.float32)
    logits_ref[...] = logits
    m = jnp.max(logits, axis=1, keepdims=True)
    ex = jnp.exp(logits - m)
    probs = ex / jnp.sum(ex, axis=1, keepdims=True)
    pmax = jnp.max(probs, axis=1, keepdims=True)
    col = lax.broadcasted_iota(jnp.int32, (S, E), 1)
    ei = jnp.min(jnp.where(probs == pmax, col, E), axis=1, keepdims=True)
    ei_ref[...] = ei
    xs_ref[...] = x * pmax
    onehot = (col == ei).astype(jnp.bfloat16)
    ranks = onehot.astype(jnp.float32)
    k = 1
    while k < S:
        shifted = jnp.concatenate(
            [jnp.zeros((k, E), jnp.float32), ranks[:S - k, :]], axis=0)
        ranks = ranks + shifted
        k *= 2
    counts = ranks[S - 1:S, :].astype(jnp.int32)
    ntiles = (counts + (TILE - 1)) // TILE
    etri = (lax.broadcasted_iota(jnp.int32, (E, E), 0)
            < lax.broadcasted_iota(jnp.int32, (E, E), 1)).astype(jnp.bfloat16)
    cum_excl = lax.dot_general(ntiles.astype(jnp.bfloat16), etri,
                               (((1,), (0,)), ((), ())),
                               preferred_element_type=jnp.float32)
    row_off = cum_excl * float(TILE)
    onehot_f = onehot.astype(jnp.float32)
    pos = jnp.sum(onehot_f * (row_off + ranks - 1.0), axis=1, keepdims=True)
    pos_ref[...] = pos.astype(jnp.int32)
    cum_incl = cum_excl + ntiles.astype(jnp.float32)
    te_ref[...] = jnp.concatenate(
        [cum_excl.astype(jnp.int32), cum_incl.astype(jnp.int32)], axis=0)


def _ffn_body(meta_ref, x_ref, wi_hbm, wo_hbm, y_ref, wi_b, wo_b, sems):
    copies = {}

    def start(j):
        e, f = divmod(j, NF)
        slot = j % NBUF
        c1 = pltpu.make_async_copy(
            wi_hbm.at[e, :, pl.ds(f * FFC, FFC)], wi_b.at[slot],
            sems.at[2 * slot])
        c2 = pltpu.make_async_copy(
            wo_hbm.at[e, pl.ds(f * FFC, FFC), :], wo_b.at[slot],
            sems.at[2 * slot + 1])
        c1.start()
        c2.start()
        copies[j] = (c1, c2)

    for j in range(NBUF - 1):
        start(j)
    for j in range(E * NF):
        e, f = divmod(j, NF)
        slot = j % NBUF
        if j + NBUF - 1 < E * NF:
            start(j + NBUF - 1)
        c1, c2 = copies.pop(j)
        c1.wait()
        c2.wait()
        t0 = meta_ref[0, e]
        t1 = meta_ref[1, e]

        def tile_body(t, carry, slot=slot, f=f):
            row = pl.multiple_of(t * TILE, TILE)
            h = jnp.maximum(
                jnp.dot(x_ref[pl.ds(row, TILE), :], wi_b[slot],
                        preferred_element_type=jnp.float32), 0.0)
            yv = jnp.dot(h, wo_b[slot], preferred_element_type=jnp.float32)
            if f == 0:
                y_ref[pl.ds(row, TILE), :] = yv
            else:
                y_ref[pl.ds(row, TILE), :] += yv
            return carry

        lax.fori_loop(t0, t1, tile_body, 0)


def _sc_scatter_body(xs_hbm, pos_hbm, out_hbm, idx_v, rows_v, sem, sem2):
    wid = lax.axis_index("s") * 2 + lax.axis_index("c")
    base = wid * CHUNK
    cp_idx = pltpu.async_copy(pos_hbm.at[pl.ds(base, CHUNK)], idx_v, sem2)
    cp_rows = pltpu.async_copy(xs_hbm.at[pl.ds(base, CHUNK)], rows_v, sem)
    cp_idx.wait()
    cp_rows.wait()
    pltpu.async_copy(rows_v, out_hbm.at[idx_v], sem).wait()


def _sc_gather_body(y_hbm, pos_hbm, out_hbm, idx_v, rows_v, sem):
    wid = lax.axis_index("s") * 2 + lax.axis_index("c")
    base = wid * CHUNK
    pltpu.sync_copy(pos_hbm.at[pl.ds(base, CHUNK)], idx_v)
    pltpu.async_copy(y_hbm.at[idx_v], rows_v, sem).wait()
    pltpu.sync_copy(rows_v, out_hbm.at[pl.ds(base, CHUNK)])


def kernel(hidden_states, router_weight, wi, wo):
    x2d = hidden_states.reshape(S, D)

    logits, ei, xs, pos, te = pl.pallas_call(
        _router_body,
        out_shape=(
            jax.ShapeDtypeStruct((S, E), jnp.float32),
            jax.ShapeDtypeStruct((S, 1), jnp.int32),
            jax.ShapeDtypeStruct((S, D), jnp.float32),
            jax.ShapeDtypeStruct((S, 1), jnp.int32),
            jax.ShapeDtypeStruct((2, E), jnp.int32),
        ),
    )(x2d, router_weight)

    pos1d = pos.reshape(S)

    mesh = plsc.VectorSubcoreMesh(core_axis_name="c", subcore_axis_name="s")
    x_sorted = pl.kernel(
        _sc_scatter_body,
        out_type=jax.ShapeDtypeStruct((NPAD, D), jnp.float32),
        mesh=mesh,
        scratch_types=[
            pltpu.VMEM((CHUNK,), jnp.int32),
            pltpu.VMEM((CHUNK, D), jnp.float32),
            pltpu.SemaphoreType.DMA,
            pltpu.SemaphoreType.DMA,
        ],
    )(xs, pos1d)

    y_sorted = pl.pallas_call(
        _ffn_body,
        grid_spec=pltpu.PrefetchScalarGridSpec(
            num_scalar_prefetch=1,
            grid=(1,),
            in_specs=[
                pl.BlockSpec((NPAD, D), lambda t, m_s: (0, 0)),
                pl.BlockSpec(memory_space=pltpu.MemorySpace.HBM),
                pl.BlockSpec(memory_space=pltpu.MemorySpace.HBM),
            ],
            out_specs=pl.BlockSpec((NPAD, D), lambda t, m_s: (0, 0)),
            scratch_shapes=[
                pltpu.VMEM((NBUF, D, FFC), jnp.float32),
                pltpu.VMEM((NBUF, FFC, D), jnp.float32),
                pltpu.SemaphoreType.DMA((2 * NBUF,)),
            ],
        ),
        out_shape=jax.ShapeDtypeStruct((NPAD, D), jnp.float32),
    )(te, x_sorted, wi, wo)

    next2d = pl.kernel(
        _sc_gather_body,
        out_type=jax.ShapeDtypeStruct((S, D), jnp.float32),
        mesh=mesh,
        scratch_types=[
            pltpu.VMEM((CHUNK,), jnp.int32),
            pltpu.VMEM((CHUNK, D), jnp.float32),
            pltpu.SemaphoreType.DMA,
        ],
    )(y_sorted, pos1d)

    return (next2d.reshape(1, S, D), logits.reshape(1, S, E), ei.reshape(1, S))


# speedup vs baseline: 1.1091x; 1.0034x over previous
import functools

import jax
import jax.numpy as jnp
from jax import lax
from jax.experimental import pallas as pl
from jax.experimental.pallas import tpu as pltpu
from jax.experimental.pallas import tpu_sc as plsc

S, D, FF, E = 2048, 1024, 2048, 8
TILE = 128
NT = 24
NPAD = NT * TILE
NW = 32
CHUNK = S // NW
HALF = CHUNK // 2
FFC = 1024
NF = FF // FFC
NBUF = 3


def _router_body(x_ref, rw_ref, logits_ref, ei_ref, xs_ref, pos_ref, te_ref):
    x = x_ref[...]
    logits = lax.dot_general(
        x, rw_ref[...], (((1,), (0,)), ((), ())),
        precision=lax.Precision.DEFAULT, preferred_element_type=jnp.float32)
    logits_ref[...] = logits
    m = jnp.max(logits, axis=1, keepdims=True)
    ex = jnp.exp(logits - m)
    probs = ex / jnp.sum(ex, axis=1, keepdims=True)
    pmax = jnp.max(probs, axis=1, keepdims=True)
    col = lax.broadcasted_iota(jnp.int32, (S, E), 1)
    ei = jnp.min(jnp.where(probs == pmax, col, E), axis=1, keepdims=True)
    ei_ref[...] = ei
    xs_ref[...] = x * pmax
    onehot = (col == ei).astype(jnp.bfloat16)
    ranks = onehot.astype(jnp.float32)
    k = 1
    while k < S:
        shifted = jnp.concatenate(
            [jnp.zeros((k, E), jnp.float32), ranks[:S - k, :]], axis=0)
        ranks = ranks + shifted
        k *= 2
    counts = ranks[S - 1:S, :].astype(jnp.int32)
    ntiles = (counts + (TILE - 1)) // TILE
    etri = (lax.broadcasted_iota(jnp.int32, (E, E), 0)
            < lax.broadcasted_iota(jnp.int32, (E, E), 1)).astype(jnp.bfloat16)
    cum_excl = lax.dot_general(ntiles.astype(jnp.bfloat16), etri,
                               (((1,), (0,)), ((), ())),
                               preferred_element_type=jnp.float32)
    row_off = cum_excl * float(TILE)
    onehot_f = onehot.astype(jnp.float32)
    pos = jnp.sum(onehot_f * (row_off + ranks - 1.0), axis=1, keepdims=True)
    pos_ref[...] = pos.astype(jnp.int32)
    cum_incl = cum_excl + ntiles.astype(jnp.float32)
    te_ref[...] = jnp.concatenate(
        [cum_excl.astype(jnp.int32), cum_incl.astype(jnp.int32)], axis=0)


def _ffn_body(meta_ref, x_ref, wi_hbm, wo_hbm, y_ref, wi_b, wo_b, sems):
    copies = {}

    def start(j):
        e, f = divmod(j, NF)
        slot = j % NBUF
        c1 = pltpu.make_async_copy(
            wi_hbm.at[e, :, pl.ds(f * FFC, FFC)], wi_b.at[slot],
            sems.at[2 * slot])
        c2 = pltpu.make_async_copy(
            wo_hbm.at[e, pl.ds(f * FFC, FFC), :], wo_b.at[slot],
            sems.at[2 * slot + 1])
        c1.start()
        c2.start()
        copies[j] = (c1, c2)

    for j in range(NBUF - 1):
        start(j)
    for j in range(E * NF):
        e, f = divmod(j, NF)
        slot = j % NBUF
        if j + NBUF - 1 < E * NF:
            start(j + NBUF - 1)
        c1, c2 = copies.pop(j)
        c1.wait()
        c2.wait()
        t0 = meta_ref[0, e]
        t1 = meta_ref[1, e]

        def tile_body(t, carry, slot=slot, f=f):
            row = pl.multiple_of(t * TILE, TILE)
            h = jnp.maximum(
                jnp.dot(x_ref[pl.ds(row, TILE), :], wi_b[slot],
                        preferred_element_type=jnp.float32), 0.0)
            yv = jnp.dot(h, wo_b[slot], preferred_element_type=jnp.float32)
            if f == 0:
                y_ref[pl.ds(row, TILE), :] = yv
            else:
                y_ref[pl.ds(row, TILE), :] += yv
            return carry

        lax.fori_loop(t0, t1, tile_body, 0)


def _sc_scatter_body(xs_hbm, pos_hbm, out_hbm, idx_a, idx_b, rows_a, rows_b,
                     sem_a, sem_b, sem_o):
    wid = lax.axis_index("s") * 2 + lax.axis_index("c")
    base = wid * CHUNK
    ia = pltpu.async_copy(pos_hbm.at[pl.ds(base, HALF)], idx_a, sem_a)
    ra = pltpu.async_copy(xs_hbm.at[pl.ds(base, HALF)], rows_a, sem_a)
    ib = pltpu.async_copy(pos_hbm.at[pl.ds(base + HALF, HALF)], idx_b, sem_b)
    rb = pltpu.async_copy(xs_hbm.at[pl.ds(base + HALF, HALF)], rows_b, sem_b)
    ia.wait()
    ra.wait()
    oa = pltpu.async_copy(rows_a, out_hbm.at[idx_a], sem_o)
    ib.wait()
    rb.wait()
    ob = pltpu.async_copy(rows_b, out_hbm.at[idx_b], sem_o)
    oa.wait()
    ob.wait()


def _sc_gather_body(y_hbm, pos_hbm, out_hbm, idx_a, idx_b, rows_a, rows_b,
                    sem_a, sem_b, sem_o):
    wid = lax.axis_index("s") * 2 + lax.axis_index("c")
    base = wid * CHUNK
    ia = pltpu.async_copy(pos_hbm.at[pl.ds(base, HALF)], idx_a, sem_a)
    ib = pltpu.async_copy(pos_hbm.at[pl.ds(base + HALF, HALF)], idx_b, sem_b)
    ia.wait()
    ga = pltpu.async_copy(y_hbm.at[idx_a], rows_a, sem_a)
    ib.wait()
    gb = pltpu.async_copy(y_hbm.at[idx_b], rows_b, sem_b)
    ga.wait()
    oa = pltpu.async_copy(rows_a, out_hbm.at[pl.ds(base, HALF)], sem_o)
    gb.wait()
    ob = pltpu.async_copy(rows_b, out_hbm.at[pl.ds(base + HALF, HALF)], sem_o)
    oa.wait()
    ob.wait()


def kernel(hidden_states, router_weight, wi, wo):
    x2d = hidden_states.reshape(S, D)

    logits, ei, xs, pos, te = pl.pallas_call(
        _router_body,
        out_shape=(
            jax.ShapeDtypeStruct((S, E), jnp.float32),
            jax.ShapeDtypeStruct((S, 1), jnp.int32),
            jax.ShapeDtypeStruct((S, D), jnp.float32),
            jax.ShapeDtypeStruct((S, 1), jnp.int32),
            jax.ShapeDtypeStruct((2, E), jnp.int32),
        ),
    )(x2d, router_weight)

    pos1d = pos.reshape(S)

    mesh = plsc.VectorSubcoreMesh(core_axis_name="c", subcore_axis_name="s")
    x_sorted = pl.kernel(
        _sc_scatter_body,
        out_type=jax.ShapeDtypeStruct((NPAD, D), jnp.float32),
        mesh=mesh,
        scratch_types=[
            pltpu.VMEM((HALF,), jnp.int32),
            pltpu.VMEM((HALF,), jnp.int32),
            pltpu.VMEM((HALF, D), jnp.float32),
            pltpu.VMEM((HALF, D), jnp.float32),
            pltpu.SemaphoreType.DMA,
            pltpu.SemaphoreType.DMA,
            pltpu.SemaphoreType.DMA,
        ],
    )(xs, pos1d)

    y_sorted = pl.pallas_call(
        _ffn_body,
        grid_spec=pltpu.PrefetchScalarGridSpec(
            num_scalar_prefetch=1,
            grid=(1,),
            in_specs=[
                pl.BlockSpec((NPAD, D), lambda t, m_s: (0, 0)),
                pl.BlockSpec(memory_space=pltpu.MemorySpace.HBM),
                pl.BlockSpec(memory_space=pltpu.MemorySpace.HBM),
            ],
            out_specs=pl.BlockSpec((NPAD, D), lambda t, m_s: (0, 0)),
            scratch_shapes=[
                pltpu.VMEM((NBUF, D, FFC), jnp.float32),
                pltpu.VMEM((NBUF, FFC, D), jnp.float32),
                pltpu.SemaphoreType.DMA((2 * NBUF,)),
            ],
        ),
        out_shape=jax.ShapeDtypeStruct((NPAD, D), jnp.float32),
    )(te, x_sorted, wi, wo)

    next2d = pl.kernel(
        _sc_gather_body,
        out_type=jax.ShapeDtypeStruct((S, D), jnp.float32),
        mesh=mesh,
        scratch_types=[
            pltpu.VMEM((HALF,), jnp.int32),
            pltpu.VMEM((HALF,), jnp.int32),
            pltpu.VMEM((HALF, D), jnp.float32),
            pltpu.VMEM((HALF, D), jnp.float32),
            pltpu.SemaphoreType.DMA,
            pltpu.SemaphoreType.DMA,
            pltpu.SemaphoreType.DMA,
        ],
    )(y_sorted, pos1d)

    return (next2d.reshape(1, S, D), logits.reshape(1, S, E), ei.reshape(1, S))
